# baseline (speedup 1.0000x reference)
import functools

import jax
import jax.numpy as jnp
from jax import lax
from jax.experimental import pallas as pl
from jax.experimental.pallas import tpu as pltpu
from jax.experimental.pallas import tpu_sc as plsc


def _xf_body(x_ref, keep_ref, xf_ref):
    xf_ref[...] = x_ref[...] * keep_ref[...]


def _build_xf(x, keep_col, R, N, D, BN):
    nblk = N // BN
    return pl.pallas_call(
        _xf_body,
        grid=(R * nblk,),
        in_specs=[
            pl.BlockSpec((BN, D), lambda b: (b % nblk, 0)),
            pl.BlockSpec((BN, 1), lambda b: (b, 0)),
        ],
        out_specs=pl.BlockSpec((BN, D), lambda b: (b, 0)),
        out_shape=jax.ShapeDtypeStruct((R * N, D), jnp.float32),
    )(x, keep_col)


def _make_sc_agg(R, N, E, D):
    L = 16
    K = 80
    NS = 16
    NC = 2
    RPC = R // NC
    CHUNKS = E // K
    BCH = 16
    DPT = 4
    NBLK = CHUNKS // BCH
    TRIPS = (NBLK + NS - 1) // NS
    REMC = CHUNKS - NBLK * BCH
    NRC = N // K
    TAIL = N - NRC * K
    RC_TRIPS = (NRC + (1 if TAIL else 0) + NS - 1) // NS

    mesh = plsc.VectorSubcoreMesh(core_axis_name="c", subcore_axis_name="s")

    def body(src_hbm, dst_hbm, off_hbm,
             xf_hbm, agg_hbm,
             spmem_agg, src_ib0, dst_ib0, src_ib1, dst_ib1, sidx16_v,
             srcr_v, dstr_v, off_v, *rest):
        sidx_vs = list(rest[0:DPT])
        dst_vs = list(rest[DPT:2 * DPT])
        rows_vs = list(rest[2 * DPT:3 * DPT])
        gsems = list(rest[3 * DPT:4 * DPT])
        ssems = list(rest[4 * DPT:5 * DPT])
        isem0, isem1 = rest[5 * DPT:5 * DPT + 2]
        gsem0 = gsems[0]
        slabs = [(src_ib0, dst_ib0, isem0), (src_ib1, dst_ib1, isem1)]
        rows_v = rows_vs[0]
        sidx_v = sidx_vs[0]
        c = lax.axis_index("c")
        s = lax.axis_index("s")

        pltpu.sync_copy(off_hbm, off_v)
        off_vec = off_v[...]

        zero16 = jnp.zeros((L,), jnp.float32)

        def run_body(k, carry_outer):
            r = c * RPC + k
            roff_vec = off_vec * r

            def zrow(i, carry):
                for t in range(D // L):
                    rows_v[i, pl.ds(t * L, L)] = zero16
                return carry

            lax.fori_loop(0, K, zrow, 0)

            def zchunk(i, carry):
                g = i * NS + s

                @pl.when(g < NRC)
                def _():
                    row0 = pl.multiple_of(g * K, 8)
                    pltpu.sync_copy(rows_v, spmem_agg.at[pl.ds(row0, K)])

                if TAIL:
                    @pl.when(g == NRC)
                    def _():
                        pltpu.sync_copy(rows_v.at[pl.ds(0, TAIL)],
                                        spmem_agg.at[pl.ds(NRC * K, TAIL)])
                return carry

            lax.fori_loop(0, RC_TRIPS, zchunk, 0)
            plsc.subcore_barrier()

            def fire(src_buf, base, b):
                for t in range(K // L):
                    sidx_vs[b][pl.ds(t * L, L)] = (
                        src_buf[pl.ds(base + t * L, L)] + roff_vec)
                return pltpu.async_copy(
                    xf_hbm.at[sidx_vs[b]], rows_vs[b], gsems[b])

            def fire_scat(dst_buf, base, b):
                for t in range(K // L):
                    dst_vs[b][pl.ds(t * L, L)] = (
                        dst_buf[pl.ds(base + t * L, L)])
                return pltpu.async_copy(rows_vs[b], spmem_agg.at[dst_vs[b]],
                                        ssems[b], add=True)

            def slab_fire(trip, p):
                e0 = pl.multiple_of((trip * NS + s) * (BCH * K), 8)
                sib, dib, isem = slabs[p]
                pltpu.async_copy(src_hbm.at[pl.ds(e0, BCH * K)], sib, isem)
                pltpu.async_copy(dst_hbm.at[pl.ds(e0, BCH * K)], dib, isem)

            def slab_wait(p):
                sib, dib, isem = slabs[p]
                pltpu.make_async_copy(
                    src_hbm.at[pl.ds(0, BCH * K)], sib, isem).wait()
                pltpu.make_async_copy(
                    dst_hbm.at[pl.ds(0, BCH * K)], dib, isem).wait()

            def process_block(sib, dib):
                gd = []
                sd = []
                for j in range(BCH):
                    if j >= DPT:
                        sd[j - DPT].wait()
                    gd.append(fire(sib, j * K, j % DPT))
                    if j >= 1:
                        gd[j - 1].wait()
                        sd.append(fire_scat(dib, (j - 1) * K, (j - 1) % DPT))
                gd[BCH - 1].wait()
                sd.append(fire_scat(dib, (BCH - 1) * K, (BCH - 1) % DPT))
                for j in range(BCH - DPT, BCH):
                    sd[j].wait()

            @pl.when(s < NBLK)
            def _():
                slab_fire(0, 0)

            def pair_body(i, carry):
                for (off_t, p) in ((0, 0), (1, 1)):
                    trip = 2 * i + off_t
                    bid = trip * NS + s
                    nxt_bid = (trip + 1) * NS + s

                    @pl.when(nxt_bid < NBLK)
                    def _():
                        slab_fire(trip + 1, 1 - p)

                    @pl.when(bid < NBLK)
                    def _():
                        slab_wait(p)
                        process_block(slabs[p][0], slabs[p][1])
                return carry

            lax.fori_loop(0, (TRIPS + 1) // 2, pair_body, 0)

            if REMC:
                @pl.when(s == 0)
                def _rem():
                    e0 = NBLK * BCH * K
                    pltpu.sync_copy(src_hbm.at[pl.ds(e0, REMC * K)], srcr_v)
                    pltpu.sync_copy(dst_hbm.at[pl.ds(e0, REMC * K)], dstr_v)
                    for q in range(REMC):
                        fire(srcr_v, q * K, 0).wait()
                        fire_scat(dstr_v, q * K, 0).wait()

            plsc.subcore_barrier()

            def srow(d_vec):
                return jnp.where(
                    d_vec < off_vec,
                    roff_vec + d_vec,
                    R * off_vec + r * (N - off_vec) + (d_vec - off_vec))

            lane = lax.iota(jnp.int32, L)

            def wchunk(i, carry):
                g = i * NS + s

                @pl.when(g < NRC)
                def _():
                    row0 = pl.multiple_of(g * K, 8)
                    pltpu.sync_copy(spmem_agg.at[pl.ds(row0, K)], rows_v)
                    for j in range(K // L):
                        sidx_v[pl.ds(j * L, L)] = srow(lane + (g * K + j * L))
                    pltpu.async_copy(rows_v, agg_hbm.at[sidx_v], gsem0).wait()

                if TAIL:
                    @pl.when(g == NRC)
                    def _():
                        pltpu.sync_copy(spmem_agg.at[pl.ds(NRC * K, TAIL)],
                                        rows_v.at[pl.ds(0, TAIL)])
                        sidx16_v[...] = srow(lane + NRC * K)
                        pltpu.async_copy(rows_v.at[pl.ds(0, TAIL)],
                                         agg_hbm.at[sidx16_v], gsem0).wait()
                return carry

            lax.fori_loop(0, RC_TRIPS, wchunk, 0)
            plsc.subcore_barrier()
            return carry_outer

        lax.fori_loop(0, RPC, run_body, 0)

    return functools.partial(
        pl.kernel,
        mesh=mesh,
        out_type=jax.ShapeDtypeStruct((R * N, D), jnp.float32),
        scratch_types=(
            [
                pltpu.VMEM_SHARED((N, D), jnp.float32),
                pltpu.VMEM((BCH * K,), jnp.int32),
                pltpu.VMEM((BCH * K,), jnp.int32),
                pltpu.VMEM((BCH * K,), jnp.int32),
                pltpu.VMEM((BCH * K,), jnp.int32),
                pltpu.VMEM((L,), jnp.int32),
                pltpu.VMEM((max(REMC, 1) * K,), jnp.int32),
                pltpu.VMEM((max(REMC, 1) * K,), jnp.int32),
                pltpu.VMEM((L,), jnp.int32),
            ]
            + [pltpu.VMEM((K,), jnp.int32) for _ in range(DPT)]
            + [pltpu.VMEM((K,), jnp.int32) for _ in range(DPT)]
            + [pltpu.VMEM((K, D), jnp.float32) for _ in range(DPT)]
            + [pltpu.SemaphoreType.DMA for _ in range(2 * DPT + 2)]
        ),
    )(body)


def _mlp_body(xf_ref, agg_ref, x_ref, w1_ref, b1_ref, w2_ref, b2_ref,
              out_ref, acc_ref, *, R):
    r = pl.program_id(1)

    @pl.when(r == 0)
    def _():
        acc_ref[...] = jnp.zeros_like(acc_ref)

    h = xf_ref[...] + agg_ref[...]
    v = jnp.dot(h, w1_ref[...], preferred_element_type=jnp.float32)
    v = v + b1_ref[...]
    v = v * jnp.tanh(jax.nn.softplus(v))
    t = jnp.dot(v, w2_ref[...], preferred_element_type=jnp.float32)
    t = t + b2_ref[...]
    acc_ref[...] += t

    @pl.when(r == R - 1)
    def _():
        out_ref[...] = acc_ref[...] * (1.0 / R) + x_ref[...]


def _mlp_mean(xf, agg, x, W1, b1, W2, b2, R, N, D, H, BN):
    nblk = N // BN
    return pl.pallas_call(
        functools.partial(_mlp_body, R=R),
        grid=(nblk, R),
        in_specs=[
            pl.BlockSpec((BN, D), lambda b, r: (r * nblk + b, 0)),
            pl.BlockSpec((BN, D), lambda b, r: (r * nblk + b, 0)),
            pl.BlockSpec((BN, D), lambda b, r: (b, 0)),
            pl.BlockSpec((D, H), lambda b, r: (0, 0)),
            pl.BlockSpec((1, H), lambda b, r: (0, 0)),
            pl.BlockSpec((H, D), lambda b, r: (0, 0)),
            pl.BlockSpec((1, D), lambda b, r: (0, 0)),
        ],
        out_specs=pl.BlockSpec((BN, D), lambda b, r: (b, 0)),
        out_shape=jax.ShapeDtypeStruct((N, D), jnp.float32),
        scratch_shapes=[pltpu.VMEM((BN, D), jnp.float32)],
    )(xf, agg, x, W1, b1, W2, b2)


def kernel(x, edge_index, drop, W1, b1, W2, b2):
    N, D = x.shape
    R = drop.shape[0]
    E = edge_index.shape[1]
    H = W1.shape[1]
    BN = 1000

    offset = (jnp.max(edge_index) + 1).astype(jnp.int32)
    keep_col = (1.0 - drop).reshape(R * N, 1)
    src = edge_index[0]
    dst = edge_index[1]
    off_arr = jnp.full((16,), offset, jnp.int32)

    xf = _build_xf(x, keep_col, R, N, D, BN)
    agg = _make_sc_agg(R, N, E, D)(src, dst, off_arr, xf)
    return _mlp_mean(xf, agg, x, W1, b1.reshape(1, H), W2, b2.reshape(1, D),
                     R, N, D, H, BN)


# speedup vs baseline: 1.0620x; 1.0620x over previous
import functools

import jax
import jax.numpy as jnp
from jax import lax
from jax.experimental import pallas as pl
from jax.experimental.pallas import tpu as pltpu
from jax.experimental.pallas import tpu_sc as plsc


def _xf_body(x_ref, keep_ref, xf_ref):
    xf_ref[...] = x_ref[...] * keep_ref[...]


def _build_xf(x, keep_col, R, N, D, BN):
    nblk = N // BN
    return pl.pallas_call(
        _xf_body,
        grid=(R * nblk,),
        in_specs=[
            pl.BlockSpec((BN, D), lambda b: (b % nblk, 0)),
            pl.BlockSpec((BN, 1), lambda b: (b, 0)),
        ],
        out_specs=pl.BlockSpec((BN, D), lambda b: (b, 0)),
        out_shape=jax.ShapeDtypeStruct((R * N, D), jnp.float32),
    )(x, keep_col)


def _make_sc_agg(R, N, E, D):
    L = 16
    K = 128
    NS = 16
    NC = 2
    RPC = R // NC
    CHUNKS = E // K
    BCH = 32
    DPT = 2
    NBLK = CHUNKS // BCH
    TRIPS = (NBLK + NS - 1) // NS
    REMC = CHUNKS - NBLK * BCH
    NRC = N // K
    TAIL = N - NRC * K
    RC_TRIPS = (NRC + (1 if TAIL else 0) + NS - 1) // NS

    mesh = plsc.VectorSubcoreMesh(core_axis_name="c", subcore_axis_name="s")

    def body(src_hbm, dst_hbm, off_hbm,
             xf_hbm, agg_hbm,
             spmem_agg, src_ib0, dst_ib0, src_ib1, dst_ib1, sidx16_v,
             srcr_v, dstr_v, off_v,
             sidx0, sidx1, dstv0, dstv1, rows0, rows1,
             gsem0, gsem1, ssem0, ssem1, isem0, isem1):
        sidx_vs = [sidx0, sidx1]
        dst_vs = [dstv0, dstv1]
        rows_vs = [rows0, rows1]
        gsems = [gsem0, gsem1]
        ssems = [ssem0, ssem1]
        slabs = [(src_ib0, dst_ib0, isem0), (src_ib1, dst_ib1, isem1)]
        rows_v = rows0
        sidx_v = sidx0
        c = lax.axis_index("c")
        s = lax.axis_index("s")

        pltpu.sync_copy(off_hbm, off_v)
        off_vec = off_v[...]

        zero16 = jnp.zeros((L,), jnp.float32)

        def run_body(k, carry_outer):
            r = c * RPC + k
            roff_vec = off_vec * r

            def zrow(i, carry):
                for t in range(K // L):
                    rows_v[i, pl.ds(t * L, L)] = zero16
                return carry

            lax.fori_loop(0, K, zrow, 0)

            def zchunk(i, carry):
                g = i * NS + s

                @pl.when(g < NRC)
                def _():
                    row0 = pl.multiple_of(g * K, 8)
                    pltpu.sync_copy(rows_v, spmem_agg.at[pl.ds(row0, K)])

                if TAIL:
                    @pl.when(g == NRC)
                    def _():
                        pltpu.sync_copy(rows_v.at[pl.ds(0, TAIL)],
                                        spmem_agg.at[pl.ds(NRC * K, TAIL)])
                return carry

            lax.fori_loop(0, RC_TRIPS, zchunk, 0)
            plsc.subcore_barrier()

            def fire(src_buf, base, b):
                for t in range(K // L):
                    sidx_vs[b][pl.ds(t * L, L)] = (
                        src_buf[pl.ds(base + t * L, L)] + roff_vec)
                return pltpu.async_copy(
                    xf_hbm.at[sidx_vs[b]], rows_vs[b], gsems[b])

            def fire_scat(dst_buf, base, b):
                for t in range(K // L):
                    dst_vs[b][pl.ds(t * L, L)] = (
                        dst_buf[pl.ds(base + t * L, L)])
                return pltpu.async_copy(rows_vs[b], spmem_agg.at[dst_vs[b]],
                                        ssems[b], add=True)

            def slab_fire(trip, p):
                e0 = pl.multiple_of((trip * NS + s) * (BCH * K), 8)
                sib, dib, isem = slabs[p]
                pltpu.async_copy(src_hbm.at[pl.ds(e0, BCH * K)], sib, isem)
                pltpu.async_copy(dst_hbm.at[pl.ds(e0, BCH * K)], dib, isem)

            def slab_wait(p):
                sib, dib, isem = slabs[p]
                pltpu.make_async_copy(
                    src_hbm.at[pl.ds(0, BCH * K)], sib, isem).wait()
                pltpu.make_async_copy(
                    dst_hbm.at[pl.ds(0, BCH * K)], dib, isem).wait()

            def process_block(sib, dib):
                gd = []
                sd = []
                for j in range(BCH):
                    if j >= DPT:
                        sd[j - DPT].wait()
                    gd.append(fire(sib, j * K, j % DPT))
                    if j >= 1:
                        gd[j - 1].wait()
                        sd.append(fire_scat(dib, (j - 1) * K, (j - 1) % DPT))
                gd[BCH - 1].wait()
                sd.append(fire_scat(dib, (BCH - 1) * K, (BCH - 1) % DPT))
                for j in range(BCH - DPT, BCH):
                    sd[j].wait()

            @pl.when(s < NBLK)
            def _():
                slab_fire(0, 0)

            def pair_body(i, carry):
                for (off_t, p) in ((0, 0), (1, 1)):
                    trip = 2 * i + off_t
                    bid = trip * NS + s
                    nxt_bid = (trip + 1) * NS + s

                    @pl.when(nxt_bid < NBLK)
                    def _():
                        slab_fire(trip + 1, 1 - p)

                    @pl.when(bid < NBLK)
                    def _():
                        slab_wait(p)
                        process_block(slabs[p][0], slabs[p][1])
                return carry

            lax.fori_loop(0, (TRIPS + 1) // 2, pair_body, 0)

            if REMC:
                @pl.when(s == 0)
                def _rem():
                    e0 = NBLK * BCH * K
                    pltpu.sync_copy(src_hbm.at[pl.ds(e0, REMC * K)], srcr_v)
                    pltpu.sync_copy(dst_hbm.at[pl.ds(e0, REMC * K)], dstr_v)
                    for q in range(REMC):
                        fire(srcr_v, q * K, 0).wait()
                        fire_scat(dstr_v, q * K, 0).wait()

            plsc.subcore_barrier()

            def srow(d_vec):
                return jnp.where(
                    d_vec < off_vec,
                    roff_vec + d_vec,
                    R * off_vec + r * (N - off_vec) + (d_vec - off_vec))

            lane = lax.iota(jnp.int32, L)

            def wchunk(i, carry):
                g = i * NS + s

                @pl.when(g < NRC)
                def _():
                    row0 = pl.multiple_of(g * K, 8)
                    pltpu.sync_copy(spmem_agg.at[pl.ds(row0, K)], rows_v)
                    for j in range(K // L):
                        sidx_v[pl.ds(j * L, L)] = srow(lane + (g * K + j * L))
                    pltpu.async_copy(rows_v, agg_hbm.at[sidx_v], gsem0).wait()

                if TAIL:
                    @pl.when(g == NRC)
                    def _():
                        pltpu.sync_copy(spmem_agg.at[pl.ds(NRC * K, TAIL)],
                                        rows_v.at[pl.ds(0, TAIL)])
                        sidx16_v[...] = srow(lane + NRC * K)
                        pltpu.async_copy(rows_v.at[pl.ds(0, TAIL)],
                                         agg_hbm.at[sidx16_v], gsem0).wait()
                return carry

            lax.fori_loop(0, RC_TRIPS, wchunk, 0)
            plsc.subcore_barrier()
            return carry_outer

        lax.fori_loop(0, RPC, run_body, 0)

    return functools.partial(
        pl.kernel,
        mesh=mesh,
        out_type=jax.ShapeDtypeStruct((R * N, D), jnp.float32),
        scratch_types=(
            [
                pltpu.VMEM_SHARED((N, D), jnp.float32),
                pltpu.VMEM((BCH * K,), jnp.int32),
                pltpu.VMEM((BCH * K,), jnp.int32),
                pltpu.VMEM((BCH * K,), jnp.int32),
                pltpu.VMEM((BCH * K,), jnp.int32),
                pltpu.VMEM((L,), jnp.int32),
                pltpu.VMEM((max(REMC, 1) * K,), jnp.int32),
                pltpu.VMEM((max(REMC, 1) * K,), jnp.int32),
                pltpu.VMEM((L,), jnp.int32),
            ]
            + [pltpu.VMEM((K,), jnp.int32) for _ in range(DPT)]
            + [pltpu.VMEM((K,), jnp.int32) for _ in range(DPT)]
            + [pltpu.VMEM((K, D), jnp.float32) for _ in range(DPT)]
            + [pltpu.SemaphoreType.DMA for _ in range(2 * DPT + 2)]
        ),
    )(body)


def _mlp_body(xf_ref, agg_ref, x_ref, w1_ref, b1_ref, w2_ref, b2_ref,
              out_ref, acc_ref, *, R):
    r = pl.program_id(1)

    @pl.when(r == 0)
    def _():
        acc_ref[...] = jnp.zeros_like(acc_ref)

    h = xf_ref[...] + agg_ref[...]
    v = jnp.dot(h, w1_ref[...], preferred_element_type=jnp.float32)
    v = v + b1_ref[...]
    v = v * jnp.tanh(jax.nn.softplus(v))
    t = jnp.dot(v, w2_ref[...], preferred_element_type=jnp.float32)
    t = t + b2_ref[...]
    acc_ref[...] += t

    @pl.when(r == R - 1)
    def _():
        out_ref[...] = acc_ref[...] * (1.0 / R) + x_ref[...]


def _mlp_mean(xf, agg, x, W1, b1, W2, b2, R, N, D, H, BN):
    nblk = N // BN
    return pl.pallas_call(
        functools.partial(_mlp_body, R=R),
        grid=(nblk, R),
        in_specs=[
            pl.BlockSpec((BN, D), lambda b, r: (r * nblk + b, 0)),
            pl.BlockSpec((BN, D), lambda b, r: (r * nblk + b, 0)),
            pl.BlockSpec((BN, D), lambda b, r: (b, 0)),
            pl.BlockSpec((D, H), lambda b, r: (0, 0)),
            pl.BlockSpec((1, H), lambda b, r: (0, 0)),
            pl.BlockSpec((H, D), lambda b, r: (0, 0)),
            pl.BlockSpec((1, D), lambda b, r: (0, 0)),
        ],
        out_specs=pl.BlockSpec((BN, D), lambda b, r: (b, 0)),
        out_shape=jax.ShapeDtypeStruct((N, D), jnp.float32),
        scratch_shapes=[pltpu.VMEM((BN, D), jnp.float32)],
    )(xf, agg, x, W1, b1, W2, b2)


def kernel(x, edge_index, drop, W1, b1, W2, b2):
    N, D = x.shape
    R = drop.shape[0]
    E = edge_index.shape[1]
    H = W1.shape[1]
    BN = 1000

    offset = (jnp.max(edge_index) + 1).astype(jnp.int32)
    keep_col = (1.0 - drop).reshape(R * N, 1)
    src = edge_index[0]
    dst = edge_index[1]
    off_arr = jnp.full((16,), offset, jnp.int32)

    xf = _build_xf(x, keep_col, R, N, D, BN)
    agg = _make_sc_agg(R, N, E, D)(src, dst, off_arr, xf)
    return _mlp_mean(xf, agg, x, W1, b1.reshape(1, H), W2, b2.reshape(1, D),
                     R, N, D, H, BN)
